# baseline (speedup 1.0000x reference)
import functools
import jax
import jax.numpy as jnp
import numpy as np
from jax import lax
from jax.experimental import pallas as pl
from jax.experimental.pallas import tpu as pltpu
from jax.experimental.pallas import tpu_sc as plsc

_NS = 16
_NC = 2
_NP = 10240
_EB = 64


def _edge_sc_body(n_rounds, src_h, dst_h, ax_h, pd_h,
                  st_out,
                  ids0, ids1, idd0, idd1, dst0, dst1,
                  ax0, ax1, pd0, pd1, st0, st_sh, sem0, sem1):
    c = lax.axis_index("c")
    sid = lax.axis_index("s")
    ep_pad = src_h.shape[0]
    per_tile = ep_pad // _NS
    n_chunks = per_tile // _EB
    rows_per = _NP // _NS
    rows0 = sid * rows_per
    banks = ((ids0, idd0, dst0, ax0, pd0, st0, sem0),
             (ids1, idd1, dst1, ax1, pd1, st0, sem1))

    def _load_idx(chunk, g, bk):
        ids, idd, dstv = bk[0], bk[1], bk[2]
        base = sid * per_tile + chunk * _EB
        pltpu.sync_copy(src_h.at[pl.ds(base, _EB)], ids)
        pltpu.sync_copy(dst_h.at[pl.ds(base, _EB)], dstv)
        for j in range(_EB // 16):
            sl = pl.ds(j * 16, 16)
            ids[sl] = ids[sl] + g * _NP
            idd[sl] = dstv[sl] + g * _NP

    def _issue(bk):
        pltpu.async_copy(ax_h.at[bk[0]], bk[3], bk[6])
        pltpu.async_copy(pd_h.at[bk[1]], bk[4], bk[6])

    def _wait(bk):
        pltpu.make_async_copy(ax_h.at[bk[0]], bk[3], bk[6]).wait()
        pltpu.make_async_copy(pd_h.at[bk[1]], bk[4], bk[6]).wait()

    for r in range(n_rounds):
        g = c * n_rounds + r
        def _z_row(i, _):
            for j in range(8):
                st0[i, pl.ds(j * 16, 16)] = jnp.zeros((16,), jnp.float32)
            return 0
        lax.fori_loop(0, _EB, _z_row, 0)
        for z in range(rows_per // _EB):
            pltpu.sync_copy(st0, st_sh.at[pl.ds(rows0 + z * _EB, _EB)])
        plsc.subcore_barrier()

        _load_idx(0, g, banks[0])
        _issue(banks[0])

        def _pair(ci, _):
            for b in range(2):
                bk = banks[b]
                nb = banks[1 - b]
                chunk = 2 * ci + b

                @pl.when(chunk + 1 < n_chunks)
                def _():
                    _load_idx(chunk + 1, g, nb)
                    _issue(nb)
                _wait(bk)
                ax_v, pd_v, st_v = bk[3], bk[4], bk[5]

                def _row(q, _):
                    for j in range(4):
                        sl = pl.ds(j * 16, 16)
                        sl2 = pl.ds(64 + j * 16, 16)
                        ee = jnp.exp(ax_v[q, sl] + pd_v[q, sl])
                        st_v[q, sl] = ee
                        st_v[q, sl2] = ee * (ax_v[q, sl2] + pd_v[q, sl2])
                    return 0
                lax.fori_loop(0, _EB, _row, 0)
                pltpu.sync_copy(st_v, st_sh.at[bk[2]], add=True)
            return 0
        lax.fori_loop(0, n_chunks // 2, _pair, 0)
        plsc.subcore_barrier()

        out0 = g * _NP + rows0
        pltpu.sync_copy(st_sh.at[pl.ds(rows0, rows_per)], st_out.at[pl.ds(out0, rows_per)])


def _edge_sc(src_p, dst_p, ax_flat, pd_flat, n_rounds):
    n_g = _NC * n_rounds
    mesh = plsc.VectorSubcoreMesh(core_axis_name="c", subcore_axis_name="s")
    f = pl.kernel(
        functools.partial(_edge_sc_body, n_rounds),
        out_type=jax.ShapeDtypeStruct((n_g * _NP, 128), jnp.float32),
        mesh=mesh,
        compiler_params=pltpu.CompilerParams(needs_layout_passes=False),
        scratch_types=(
            [pltpu.VMEM((_EB,), jnp.int32) for _ in range(6)]
            + [pltpu.VMEM((_EB, 128), jnp.float32) for _ in range(5)]
            + [pltpu.VMEM_SHARED((_NP, 128), jnp.float32),
               pltpu.SemaphoreType.DMA, pltpu.SemaphoreType.DMA]
        ),
    )
    return f(src_p, dst_p, ax_flat, pd_flat)


_BN = 512
_NBLK = _NP // _BN


def _dot(a, b):
    return jax.lax.dot_general(a, b, (((1,), (0,)), ((), ())),
                               preferred_element_type=jnp.float32)


def _ln(x, g, b, eps=1e-5):
    mu = jnp.mean(x, axis=-1, keepdims=True)
    var = jnp.mean((x - mu) ** 2, axis=-1, keepdims=True)
    return (x - mu) / jnp.sqrt(var + eps) * g + b


def _conv_max_body(x_r, p_r, wsrc_r, wpos_r, bpos_r, m_o, rm):
    i = pl.program_id(1)
    P = _dot(p_r[...], wpos_r[0])
    A = -_dot(x_r[...], wsrc_r[0]) - P
    PB = P + bpos_r[0]
    ca = jnp.max(A, axis=0, keepdims=True)
    cp = jnp.max(PB, axis=0, keepdims=True)

    @pl.when(i == 0)
    def _():
        rm[0:1] = ca
        rm[1:2] = cp

    @pl.when(i > 0)
    def _():
        rm[0:1] = jnp.maximum(rm[0:1], ca)
        rm[1:2] = jnp.maximum(rm[1:2], cp)

    @pl.when(i == _NBLK - 1)
    def _():
        m_o[0] = rm[0:1] + rm[1:2]


def _conv_pre_body(x_r, p_r, wsrc_r, wlin_r, blin_r, wpos_r, bpos_r, m_r,
                   ax_o, pd_o):
    x = x_r[...]
    p3 = p_r[...]
    P = _dot(p3, wpos_r[0])
    A = -_dot(x, wsrc_r[0]) - P - m_r[0]
    XV = _dot(x, wlin_r[0]) + blin_r[0] - P
    PB = P + bpos_r[0]
    ax_o[0] = jnp.concatenate([A, XV], axis=-1)
    pd_o[0] = jnp.concatenate([PB, PB], axis=-1)


def _conv_pre(xp, posp, wsrc_g, wlin_g, blin_g, wpos_g, bpos_g):
    F = xp.shape[1]
    n_g = wsrc_g.shape[0]
    m = pl.pallas_call(
        _conv_max_body,
        grid=(n_g, _NBLK),
        in_specs=[
            pl.BlockSpec((_BN, F), lambda g, i: (i, 0)),
            pl.BlockSpec((_BN, 8), lambda g, i: (i, 0)),
            pl.BlockSpec((1, F, 64), lambda g, i: (g, 0, 0)),
            pl.BlockSpec((1, 8, 64), lambda g, i: (g, 0, 0)),
            pl.BlockSpec((1, 1, 64), lambda g, i: (g, 0, 0)),
        ],
        out_specs=pl.BlockSpec((1, 1, 64), lambda g, i: (g, 0, 0)),
        out_shape=jax.ShapeDtypeStruct((n_g, 1, 64), jnp.float32),
        scratch_shapes=[pltpu.VMEM((2, 64), jnp.float32)],
    )(xp, posp, wsrc_g, wpos_g, bpos_g)
    ax, pd = pl.pallas_call(
        _conv_pre_body,
        grid=(n_g, _NBLK),
        in_specs=[
            pl.BlockSpec((_BN, F), lambda g, i: (i, 0)),
            pl.BlockSpec((_BN, 8), lambda g, i: (i, 0)),
            pl.BlockSpec((1, F, 64), lambda g, i: (g, 0, 0)),
            pl.BlockSpec((1, F, 64), lambda g, i: (g, 0, 0)),
            pl.BlockSpec((1, 1, 64), lambda g, i: (g, 0, 0)),
            pl.BlockSpec((1, 8, 64), lambda g, i: (g, 0, 0)),
            pl.BlockSpec((1, 1, 64), lambda g, i: (g, 0, 0)),
            pl.BlockSpec((1, 1, 64), lambda g, i: (g, 0, 0)),
        ],
        out_specs=[
            pl.BlockSpec((1, _BN, 128), lambda g, i: (g, i, 0)),
            pl.BlockSpec((1, _BN, 128), lambda g, i: (g, i, 0)),
        ],
        out_shape=[
            jax.ShapeDtypeStruct((n_g, _NP, 128), jnp.float32),
            jax.ShapeDtypeStruct((n_g, _NP, 128), jnp.float32),
        ],
    )(xp, posp, wsrc_g, wlin_g, blin_g, wpos_g, bpos_g, m)
    return ax, pd


def _conv_post_body(st_r, w1_r, b1_r, w2_r, b2_r, y_o, *, n_g):
    st = st_r[...]
    acc = None
    for g in range(n_g):
        S = st[g, :, :64]
        T = st[g, :, 64:]
        R = T / (S + 1e-16)
        pa = _dot(R, w1_r[pl.ds(g * 64, 64), :])
        acc = pa if acc is None else acc + pa
    h = jax.nn.relu(acc + b1_r[...])
    y_o[...] = _dot(h, w2_r[...]) + b2_r[...]


def _conv_post(st, w1, b1, w2, b2):
    n_g, _, _ = st.shape
    Hc, Ho = w2.shape
    return pl.pallas_call(
        functools.partial(_conv_post_body, n_g=n_g),
        grid=(_NBLK,),
        in_specs=[
            pl.BlockSpec((n_g, _BN, 128), lambda i: (0, i, 0)),
            pl.BlockSpec((Hc, Hc), lambda i: (0, 0)),
            pl.BlockSpec((1, Hc), lambda i: (0, 0)),
            pl.BlockSpec((Hc, Ho), lambda i: (0, 0)),
            pl.BlockSpec((1, Ho), lambda i: (0, 0)),
        ],
        out_specs=pl.BlockSpec((_BN, Ho), lambda i: (i, 0)),
        out_shape=jax.ShapeDtypeStruct((_NP, Ho), jnp.float32),
    )(st, w1, b1, w2, b2)


def _attn_pre_body(x_r, p_r, g1_r, b1_r, pe1w_r, pe1b_r, pe2w_r, pe2b_r,
                   wct_r, wcb_r, bc_r, wq_r, bq_r, wk_r, bk_r, wv_r, bv_r,
                   q_o, k_o, v_o):
    x = x_r[...]
    xn = _ln(x, g1_r[...], b1_r[...])
    pe = jax.nn.relu(_dot(p_r[...], pe1w_r[...]) + pe1b_r[...])
    pe = _dot(pe, pe2w_r[...]) + pe2b_r[...]
    xc = _dot(xn, wct_r[...]) + _dot(pe, wcb_r[...]) + bc_r[...]
    q_o[...] = _dot(xc, wq_r[...]) + bq_r[...]
    k_o[...] = _dot(xc, wk_r[...]) + bk_r[...]
    v_o[...] = _dot(xc, wv_r[...]) + bv_r[...]


def _attn_pre(xp, posp, p):
    dim = xp.shape[1]
    pe_d = p['pe2']['W'].shape[1]
    wct = p['comb']['W'][:dim]
    wcb = p['comb']['W'][dim:]
    args = (xp, posp, p['g1'][None], p['b1'][None],
            jnp.pad(p['pe1']['W'], ((0, 5), (0, 0))), p['pe1']['b'][None],
            p['pe2']['W'], p['pe2']['b'][None],
            wct, wcb, p['comb']['b'][None],
            p['q']['W'], p['q']['b'][None], p['k']['W'], p['k']['b'][None],
            p['v']['W'], p['v']['b'][None])
    specs = [
        pl.BlockSpec((_BN, dim), lambda i: (i, 0)),
        pl.BlockSpec((_BN, 8), lambda i: (i, 0)),
        pl.BlockSpec((1, dim), lambda i: (0, 0)),
        pl.BlockSpec((1, dim), lambda i: (0, 0)),
        pl.BlockSpec((8, pe_d), lambda i: (0, 0)),
        pl.BlockSpec((1, pe_d), lambda i: (0, 0)),
        pl.BlockSpec((pe_d, pe_d), lambda i: (0, 0)),
        pl.BlockSpec((1, pe_d), lambda i: (0, 0)),
        pl.BlockSpec((dim, dim), lambda i: (0, 0)),
        pl.BlockSpec((pe_d, dim), lambda i: (0, 0)),
        pl.BlockSpec((1, dim), lambda i: (0, 0)),
        pl.BlockSpec((dim, dim), lambda i: (0, 0)),
        pl.BlockSpec((1, dim), lambda i: (0, 0)),
        pl.BlockSpec((dim, dim), lambda i: (0, 0)),
        pl.BlockSpec((1, dim), lambda i: (0, 0)),
        pl.BlockSpec((dim, dim), lambda i: (0, 0)),
        pl.BlockSpec((1, dim), lambda i: (0, 0)),
    ]
    out = jax.ShapeDtypeStruct((_NP, dim), jnp.float32)
    return pl.pallas_call(
        _attn_pre_body,
        grid=(_NBLK,),
        in_specs=specs,
        out_specs=[pl.BlockSpec((_BN, dim), lambda i: (i, 0))] * 3,
        out_shape=[out, out, out],
    )(*args)


def _attn_body(q_ref, k_ref, v_ref, o_ref, *, n_valid, scale):
    q = (q_ref[0] * scale).astype(jnp.bfloat16)
    k = k_ref[0].astype(jnp.bfloat16)
    v = v_ref[0].astype(jnp.bfloat16)
    s = jax.lax.dot_general(q, k, (((1,), (1,)), ((), ())),
                            preferred_element_type=jnp.float32)
    col = jax.lax.broadcasted_iota(jnp.int32, s.shape, 1)
    s = jnp.where(col < n_valid, s, -1e30)
    m = jnp.max(s, axis=1, keepdims=True)
    pf = jnp.exp(s - m)
    den = jnp.sum(pf, axis=1, keepdims=True)
    p = pf.astype(jnp.bfloat16)
    o = jax.lax.dot_general(p, v, (((1,), (0,)), ((), ())),
                            preferred_element_type=jnp.float32)
    o_ref[0] = o / den


def _attention(q, k, v, n_valid, num_heads=4, bq=512):
    _, Np, hd = q.shape
    scale = 1.0 / np.sqrt(hd)
    return pl.pallas_call(
        functools.partial(_attn_body, n_valid=n_valid, scale=scale),
        grid=(num_heads, Np // bq),
        in_specs=[
            pl.BlockSpec((1, bq, hd), lambda h, i: (h, i, 0)),
            pl.BlockSpec((1, Np, hd), lambda h, i: (h, 0, 0)),
            pl.BlockSpec((1, Np, hd), lambda h, i: (h, 0, 0)),
        ],
        out_specs=pl.BlockSpec((1, bq, hd), lambda h, i: (h, i, 0)),
        out_shape=jax.ShapeDtypeStruct((num_heads, Np, hd), jnp.float32),
    )(q, k, v)


def _attn_post_body(o_r, x_r, wo_r, bo_r, g2_r, b2_r, y_o):
    out = _dot(o_r[...], wo_r[...]) + bo_r[...] + x_r[...]
    y_o[...] = _ln(out, g2_r[...], b2_r[...])


def _attn_post(o, xp, p):
    dim = xp.shape[1]
    return pl.pallas_call(
        _attn_post_body,
        grid=(_NBLK,),
        in_specs=[
            pl.BlockSpec((_BN, dim), lambda i: (i, 0)),
            pl.BlockSpec((_BN, dim), lambda i: (i, 0)),
            pl.BlockSpec((dim, dim), lambda i: (0, 0)),
            pl.BlockSpec((1, dim), lambda i: (0, 0)),
            pl.BlockSpec((1, dim), lambda i: (0, 0)),
            pl.BlockSpec((1, dim), lambda i: (0, 0)),
        ],
        out_specs=pl.BlockSpec((_BN, dim), lambda i: (i, 0)),
        out_shape=jax.ShapeDtypeStruct((_NP, dim), jnp.float32),
    )(o, xp, p['o']['W'], p['o']['b'][None], p['g2'][None], p['b2'][None])


def _multi_head_conv(xp, posp, src, dst, p, n_valid):
    Wsrc = jnp.concatenate([hp['src']['W'] for hp in p['heads']], axis=1)
    Wlin = jnp.concatenate([hp['lin']['W'] for hp in p['heads']], axis=1)
    blin = jnp.concatenate([hp['lin']['b'] for hp in p['heads']], axis=0)
    Wpos = jnp.concatenate([hp['pos']['W'] for hp in p['heads']], axis=1)
    bpos = jnp.concatenate([hp['pos']['b'] for hp in p['heads']], axis=0)
    F = xp.shape[1]
    Hc = Wsrc.shape[1]
    n_g = Hc // 64
    n_rounds = n_g // _NC
    if Wsrc.shape[0] != F:
        Wsrc = jnp.pad(Wsrc, ((0, F - Wsrc.shape[0]), (0, 0)))
        Wlin = jnp.pad(Wlin, ((0, F - Wlin.shape[0]), (0, 0)))

    def _grp(w):
        return w.reshape(w.shape[0], n_g, 64).transpose(1, 0, 2)
    ax, pd = _conv_pre(xp, posp, _grp(Wsrc), _grp(Wlin),
                       blin.reshape(n_g, 1, 64),
                       _grp(jnp.pad(Wpos, ((0, 5), (0, 0)))),
                       bpos.reshape(n_g, 1, 64))
    ST = _edge_sc(src, dst, ax.reshape(n_g * _NP, 128),
                  pd.reshape(n_g * _NP, 128), n_rounds)
    return _conv_post(ST.reshape(n_g, _NP, 128), p['p1']['W'],
                      p['p1']['b'][None], p['p2']['W'], p['p2']['b'][None])


def _pos_attn(xp, posp, p, n_valid, num_heads=4):
    q, k, v = _attn_pre(xp, posp, p)
    Np, dim = q.shape
    hd = dim // num_heads

    def _heads(t):
        return t.reshape(Np, num_heads, hd).transpose(1, 0, 2)
    o = _attention(_heads(q), _heads(k), _heads(v), n_valid, num_heads)
    o = o.transpose(1, 0, 2).reshape(Np, dim)
    return _attn_post(o, xp, p)


def kernel(pos, attr, edge_index, params):
    N = pos.shape[0]
    loop = jnp.arange(N, dtype=edge_index.dtype)
    E = edge_index.shape[1] + N
    ep_pad = -(-E // (_NS * _EB * 2)) * (_NS * _EB * 2)
    src = jnp.concatenate([edge_index[0], loop,
                           jnp.zeros((ep_pad - E,), edge_index.dtype)])
    dst = jnp.concatenate([edge_index[1], loop,
                           jnp.full((ep_pad - E,), N, edge_index.dtype)])
    posp = jnp.pad(pos, ((0, _NP - N), (0, 5)))
    xp = jnp.pad(attr, ((0, _NP - N), (0, 5)))
    x = _multi_head_conv(xp, posp, src, dst, params['pt1'], N)
    x = _pos_attn(x, posp, params['attn'], N)
    x = _multi_head_conv(x, posp, src, dst, params['pt2'], N)
    return x[:N]


# speedup vs baseline: 1.1157x; 1.1157x over previous
import functools
import jax
import jax.numpy as jnp
import numpy as np
from jax import lax
from jax.experimental import pallas as pl
from jax.experimental.pallas import tpu as pltpu
from jax.experimental.pallas import tpu_sc as plsc

_NS = 16
_NC = 2
_NP = 10240
_EB = 64


def _edge_sc_body(n_rounds, src_h, dst_h, ax_h, pd_h,
                  st_out,
                  ids0, ids1, idd0, idd1, dst0, dst1,
                  ax0, ax1, pd0, pd1, st0, st_sh, sem0, sem1):
    c = lax.axis_index("c")
    sid = lax.axis_index("s")
    ep_pad = src_h.shape[0]
    per_tile = ep_pad // _NS
    n_chunks = per_tile // _EB
    rows_per = _NP // _NS
    rows0 = sid * rows_per
    banks = ((ids0, idd0, dst0, ax0, pd0, st0, sem0),
             (ids1, idd1, dst1, ax1, pd1, st0, sem1))

    def _load_idx(chunk, g, bk):
        ids, idd, dstv = bk[0], bk[1], bk[2]
        base = sid * per_tile + chunk * _EB
        pltpu.sync_copy(src_h.at[pl.ds(base, _EB)], ids)
        pltpu.sync_copy(dst_h.at[pl.ds(base, _EB)], dstv)
        for j in range(_EB // 16):
            sl = pl.ds(j * 16, 16)
            ids[sl] = ids[sl] + g * _NP
            idd[sl] = dstv[sl] + g * _NP

    def _issue(bk):
        pltpu.async_copy(ax_h.at[bk[0]], bk[3], bk[6])
        pltpu.async_copy(pd_h.at[bk[1]], bk[4], bk[6])

    def _wait(bk):
        pltpu.make_async_copy(ax_h.at[bk[0]], bk[3], bk[6]).wait()
        pltpu.make_async_copy(pd_h.at[bk[1]], bk[4], bk[6]).wait()

    for r in range(n_rounds):
        g = c * n_rounds + r
        def _z_row(i, _):
            for j in range(8):
                st0[i, pl.ds(j * 16, 16)] = jnp.zeros((16,), jnp.float32)
            return 0
        lax.fori_loop(0, _EB, _z_row, 0)
        for z in range(rows_per // _EB):
            pltpu.sync_copy(st0, st_sh.at[pl.ds(rows0 + z * _EB, _EB)])
        plsc.subcore_barrier()

        _load_idx(0, g, banks[0])
        _issue(banks[0])

        def _pair(ci, _):
            for b in range(2):
                bk = banks[b]
                nb = banks[1 - b]
                chunk = 2 * ci + b

                @pl.when(chunk + 1 < n_chunks)
                def _():
                    _load_idx(chunk + 1, g, nb)
                    _issue(nb)
                _wait(bk)
                ax_v, pd_v, st_v = bk[3], bk[4], bk[5]

                def _row(q, _):
                    for j in range(4):
                        sl = pl.ds(j * 16, 16)
                        sl2 = pl.ds(64 + j * 16, 16)
                        ee = jnp.exp(ax_v[q, sl] + pd_v[q, sl])
                        st_v[q, sl] = ee
                        st_v[q, sl2] = ee * (ax_v[q, sl2] + pd_v[q, sl2])
                    return 0
                lax.fori_loop(0, _EB, _row, 0)
                pltpu.sync_copy(st_v, st_sh.at[bk[2]], add=True)
            return 0
        lax.fori_loop(0, n_chunks // 2, _pair, 0)
        plsc.subcore_barrier()

        out0 = g * _NP + rows0
        pltpu.sync_copy(st_sh.at[pl.ds(rows0, rows_per)], st_out.at[pl.ds(out0, rows_per)])


def _edge_sc(src_p, dst_p, ax_flat, pd_flat, n_rounds):
    n_g = _NC * n_rounds
    mesh = plsc.VectorSubcoreMesh(core_axis_name="c", subcore_axis_name="s")
    f = pl.kernel(
        functools.partial(_edge_sc_body, n_rounds),
        out_type=jax.ShapeDtypeStruct((n_g * _NP, 128), jnp.float32),
        mesh=mesh,
        compiler_params=pltpu.CompilerParams(needs_layout_passes=False),
        scratch_types=(
            [pltpu.VMEM((_EB,), jnp.int32) for _ in range(6)]
            + [pltpu.VMEM((_EB, 128), jnp.float32) for _ in range(5)]
            + [pltpu.VMEM_SHARED((_NP, 128), jnp.float32),
               pltpu.SemaphoreType.DMA, pltpu.SemaphoreType.DMA]
        ),
    )
    return f(src_p, dst_p, ax_flat, pd_flat)


_BN = 512
_NBLK = _NP // _BN


def _dot(a, b):
    return jax.lax.dot_general(a, b, (((1,), (0,)), ((), ())),
                               preferred_element_type=jnp.float32)


def _ln(x, g, b, eps=1e-5):
    mu = jnp.mean(x, axis=-1, keepdims=True)
    var = jnp.mean((x - mu) ** 2, axis=-1, keepdims=True)
    return (x - mu) / jnp.sqrt(var + eps) * g + b


def _conv_max_body(x_r, p_r, wsrc_r, wpos_r, bpos_r, m_o, rm):
    i = pl.program_id(1)
    P = _dot(p_r[...], wpos_r[0])
    A = -_dot(x_r[...], wsrc_r[0]) - P
    PB = P + bpos_r[0]
    ca = jnp.max(A, axis=0, keepdims=True)
    cp = jnp.max(PB, axis=0, keepdims=True)

    @pl.when(i == 0)
    def _():
        rm[0:1] = ca
        rm[1:2] = cp

    @pl.when(i > 0)
    def _():
        rm[0:1] = jnp.maximum(rm[0:1], ca)
        rm[1:2] = jnp.maximum(rm[1:2], cp)

    @pl.when(i == _NBLK - 1)
    def _():
        m_o[0] = rm[0:1] + rm[1:2]


def _conv_pre_body(x_r, p_r, wsrc_r, wlin_r, blin_r, wpos_r, bpos_r, m_r,
                   ax_o, pd_o):
    x = x_r[...]
    p3 = p_r[...]
    P = _dot(p3, wpos_r[0])
    A = -_dot(x, wsrc_r[0]) - P - m_r[0]
    XV = _dot(x, wlin_r[0]) + blin_r[0] - P
    PB = P + bpos_r[0]
    ax_o[0] = jnp.concatenate([A, XV], axis=-1)
    pd_o[0] = jnp.concatenate([PB, PB], axis=-1)


def _conv_pre(xp, posp, wsrc_g, wlin_g, blin_g, wpos_g, bpos_g):
    F = xp.shape[1]
    n_g = wsrc_g.shape[0]
    m = pl.pallas_call(
        _conv_max_body,
        grid=(n_g, _NBLK),
        in_specs=[
            pl.BlockSpec((_BN, F), lambda g, i: (i, 0)),
            pl.BlockSpec((_BN, 8), lambda g, i: (i, 0)),
            pl.BlockSpec((1, F, 64), lambda g, i: (g, 0, 0)),
            pl.BlockSpec((1, 8, 64), lambda g, i: (g, 0, 0)),
            pl.BlockSpec((1, 1, 64), lambda g, i: (g, 0, 0)),
        ],
        out_specs=pl.BlockSpec((1, 1, 64), lambda g, i: (g, 0, 0)),
        out_shape=jax.ShapeDtypeStruct((n_g, 1, 64), jnp.float32),
        scratch_shapes=[pltpu.VMEM((2, 64), jnp.float32)],
    )(xp, posp, wsrc_g, wpos_g, bpos_g)
    ax, pd = pl.pallas_call(
        _conv_pre_body,
        grid=(n_g, _NBLK),
        in_specs=[
            pl.BlockSpec((_BN, F), lambda g, i: (i, 0)),
            pl.BlockSpec((_BN, 8), lambda g, i: (i, 0)),
            pl.BlockSpec((1, F, 64), lambda g, i: (g, 0, 0)),
            pl.BlockSpec((1, F, 64), lambda g, i: (g, 0, 0)),
            pl.BlockSpec((1, 1, 64), lambda g, i: (g, 0, 0)),
            pl.BlockSpec((1, 8, 64), lambda g, i: (g, 0, 0)),
            pl.BlockSpec((1, 1, 64), lambda g, i: (g, 0, 0)),
            pl.BlockSpec((1, 1, 64), lambda g, i: (g, 0, 0)),
        ],
        out_specs=[
            pl.BlockSpec((1, _BN, 128), lambda g, i: (g, i, 0)),
            pl.BlockSpec((1, _BN, 128), lambda g, i: (g, i, 0)),
        ],
        out_shape=[
            jax.ShapeDtypeStruct((n_g, _NP, 128), jnp.float32),
            jax.ShapeDtypeStruct((n_g, _NP, 128), jnp.float32),
        ],
    )(xp, posp, wsrc_g, wlin_g, blin_g, wpos_g, bpos_g, m)
    return ax, pd


def _conv_post_body(st_r, w1_r, b1_r, w2_r, b2_r, y_o, *, n_g):
    st = st_r[...]
    acc = None
    for g in range(n_g):
        S = st[g, :, :64]
        T = st[g, :, 64:]
        R = T / (S + 1e-16)
        pa = _dot(R, w1_r[pl.ds(g * 64, 64), :])
        acc = pa if acc is None else acc + pa
    h = jax.nn.relu(acc + b1_r[...])
    y_o[...] = _dot(h, w2_r[...]) + b2_r[...]


def _conv_post(st, w1, b1, w2, b2):
    n_g, _, _ = st.shape
    Hc, Ho = w2.shape
    return pl.pallas_call(
        functools.partial(_conv_post_body, n_g=n_g),
        grid=(_NBLK,),
        in_specs=[
            pl.BlockSpec((n_g, _BN, 128), lambda i: (0, i, 0)),
            pl.BlockSpec((Hc, Hc), lambda i: (0, 0)),
            pl.BlockSpec((1, Hc), lambda i: (0, 0)),
            pl.BlockSpec((Hc, Ho), lambda i: (0, 0)),
            pl.BlockSpec((1, Ho), lambda i: (0, 0)),
        ],
        out_specs=pl.BlockSpec((_BN, Ho), lambda i: (i, 0)),
        out_shape=jax.ShapeDtypeStruct((_NP, Ho), jnp.float32),
    )(st, w1, b1, w2, b2)


def _attn_pre_body(x_r, p_r, g1_r, b1_r, pe1w_r, pe1b_r, pe2w_r, pe2b_r,
                   wct_r, wcb_r, bc_r, wq_r, bq_r, wk_r, bk_r, wv_r, bv_r,
                   q_o, k_o, v_o):
    x = x_r[...]
    xn = _ln(x, g1_r[...], b1_r[...])
    pe = jax.nn.relu(_dot(p_r[...], pe1w_r[...]) + pe1b_r[...])
    pe = _dot(pe, pe2w_r[...]) + pe2b_r[...]
    xc = _dot(xn, wct_r[...]) + _dot(pe, wcb_r[...]) + bc_r[...]
    q_o[...] = _dot(xc, wq_r[...]) + bq_r[...]
    k_o[...] = _dot(xc, wk_r[...]) + bk_r[...]
    v_o[...] = _dot(xc, wv_r[...]) + bv_r[...]


def _attn_pre(xp, posp, p):
    dim = xp.shape[1]
    pe_d = p['pe2']['W'].shape[1]
    wct = p['comb']['W'][:dim]
    wcb = p['comb']['W'][dim:]
    args = (xp, posp, p['g1'][None], p['b1'][None],
            jnp.pad(p['pe1']['W'], ((0, 5), (0, 0))), p['pe1']['b'][None],
            p['pe2']['W'], p['pe2']['b'][None],
            wct, wcb, p['comb']['b'][None],
            p['q']['W'], p['q']['b'][None], p['k']['W'], p['k']['b'][None],
            p['v']['W'], p['v']['b'][None])
    specs = [
        pl.BlockSpec((_BN, dim), lambda i: (i, 0)),
        pl.BlockSpec((_BN, 8), lambda i: (i, 0)),
        pl.BlockSpec((1, dim), lambda i: (0, 0)),
        pl.BlockSpec((1, dim), lambda i: (0, 0)),
        pl.BlockSpec((8, pe_d), lambda i: (0, 0)),
        pl.BlockSpec((1, pe_d), lambda i: (0, 0)),
        pl.BlockSpec((pe_d, pe_d), lambda i: (0, 0)),
        pl.BlockSpec((1, pe_d), lambda i: (0, 0)),
        pl.BlockSpec((dim, dim), lambda i: (0, 0)),
        pl.BlockSpec((pe_d, dim), lambda i: (0, 0)),
        pl.BlockSpec((1, dim), lambda i: (0, 0)),
        pl.BlockSpec((dim, dim), lambda i: (0, 0)),
        pl.BlockSpec((1, dim), lambda i: (0, 0)),
        pl.BlockSpec((dim, dim), lambda i: (0, 0)),
        pl.BlockSpec((1, dim), lambda i: (0, 0)),
        pl.BlockSpec((dim, dim), lambda i: (0, 0)),
        pl.BlockSpec((1, dim), lambda i: (0, 0)),
    ]
    out = jax.ShapeDtypeStruct((_NP, dim), jnp.float32)
    return pl.pallas_call(
        _attn_pre_body,
        grid=(_NBLK,),
        in_specs=specs,
        out_specs=[pl.BlockSpec((_BN, dim), lambda i: (i, 0))] * 3,
        out_shape=[out, out, out],
    )(*args)


def _attn_body(q_ref, k_ref, v_ref, o_ref, *, n_valid, scale):
    q = (q_ref[0] * scale).astype(jnp.bfloat16)
    k = k_ref[0].astype(jnp.bfloat16)
    v = v_ref[0].astype(jnp.bfloat16)
    s = jax.lax.dot_general(q, k, (((1,), (1,)), ((), ())),
                            preferred_element_type=jnp.float32)
    col = jax.lax.broadcasted_iota(jnp.int32, s.shape, 1)
    s = jnp.where(col < n_valid, s, -1e30)
    m = jnp.max(s, axis=1, keepdims=True)
    pf = jnp.exp(s - m)
    den = jnp.sum(pf, axis=1, keepdims=True)
    p = pf.astype(jnp.bfloat16)
    o = jax.lax.dot_general(p, v, (((1,), (0,)), ((), ())),
                            preferred_element_type=jnp.float32)
    o_ref[0] = o / den


def _attention(q, k, v, n_valid, num_heads=4, bq=128):
    _, Np, hd = q.shape
    scale = 1.0 / np.sqrt(hd)
    return pl.pallas_call(
        functools.partial(_attn_body, n_valid=n_valid, scale=scale),
        grid=(num_heads, Np // bq),
        in_specs=[
            pl.BlockSpec((1, bq, hd), lambda h, i: (h, i, 0)),
            pl.BlockSpec((1, Np, hd), lambda h, i: (h, 0, 0)),
            pl.BlockSpec((1, Np, hd), lambda h, i: (h, 0, 0)),
        ],
        out_specs=pl.BlockSpec((1, bq, hd), lambda h, i: (h, i, 0)),
        out_shape=jax.ShapeDtypeStruct((num_heads, Np, hd), jnp.float32),
    )(q, k, v)


def _attn_post_body(o_r, x_r, wo_r, bo_r, g2_r, b2_r, y_o):
    out = _dot(o_r[...], wo_r[...]) + bo_r[...] + x_r[...]
    y_o[...] = _ln(out, g2_r[...], b2_r[...])


def _attn_post(o, xp, p):
    dim = xp.shape[1]
    return pl.pallas_call(
        _attn_post_body,
        grid=(_NBLK,),
        in_specs=[
            pl.BlockSpec((_BN, dim), lambda i: (i, 0)),
            pl.BlockSpec((_BN, dim), lambda i: (i, 0)),
            pl.BlockSpec((dim, dim), lambda i: (0, 0)),
            pl.BlockSpec((1, dim), lambda i: (0, 0)),
            pl.BlockSpec((1, dim), lambda i: (0, 0)),
            pl.BlockSpec((1, dim), lambda i: (0, 0)),
        ],
        out_specs=pl.BlockSpec((_BN, dim), lambda i: (i, 0)),
        out_shape=jax.ShapeDtypeStruct((_NP, dim), jnp.float32),
    )(o, xp, p['o']['W'], p['o']['b'][None], p['g2'][None], p['b2'][None])


def _multi_head_conv(xp, posp, src, dst, p, n_valid):
    Wsrc = jnp.concatenate([hp['src']['W'] for hp in p['heads']], axis=1)
    Wlin = jnp.concatenate([hp['lin']['W'] for hp in p['heads']], axis=1)
    blin = jnp.concatenate([hp['lin']['b'] for hp in p['heads']], axis=0)
    Wpos = jnp.concatenate([hp['pos']['W'] for hp in p['heads']], axis=1)
    bpos = jnp.concatenate([hp['pos']['b'] for hp in p['heads']], axis=0)
    F = xp.shape[1]
    Hc = Wsrc.shape[1]
    n_g = Hc // 64
    n_rounds = n_g // _NC
    if Wsrc.shape[0] != F:
        Wsrc = jnp.pad(Wsrc, ((0, F - Wsrc.shape[0]), (0, 0)))
        Wlin = jnp.pad(Wlin, ((0, F - Wlin.shape[0]), (0, 0)))

    def _grp(w):
        return w.reshape(w.shape[0], n_g, 64).transpose(1, 0, 2)
    ax, pd = _conv_pre(xp, posp, _grp(Wsrc), _grp(Wlin),
                       blin.reshape(n_g, 1, 64),
                       _grp(jnp.pad(Wpos, ((0, 5), (0, 0)))),
                       bpos.reshape(n_g, 1, 64))
    ST = _edge_sc(src, dst, ax.reshape(n_g * _NP, 128),
                  pd.reshape(n_g * _NP, 128), n_rounds)
    return _conv_post(ST.reshape(n_g, _NP, 128), p['p1']['W'],
                      p['p1']['b'][None], p['p2']['W'], p['p2']['b'][None])


def _pos_attn(xp, posp, p, n_valid, num_heads=4):
    q, k, v = _attn_pre(xp, posp, p)
    Np, dim = q.shape
    hd = dim // num_heads

    def _heads(t):
        return t.reshape(Np, num_heads, hd).transpose(1, 0, 2)
    o = _attention(_heads(q), _heads(k), _heads(v), n_valid, num_heads)
    o = o.transpose(1, 0, 2).reshape(Np, dim)
    return _attn_post(o, xp, p)


def kernel(pos, attr, edge_index, params):
    N = pos.shape[0]
    loop = jnp.arange(N, dtype=edge_index.dtype)
    E = edge_index.shape[1] + N
    ep_pad = -(-E // (_NS * _EB * 2)) * (_NS * _EB * 2)
    src = jnp.concatenate([edge_index[0], loop,
                           jnp.zeros((ep_pad - E,), edge_index.dtype)])
    dst = jnp.concatenate([edge_index[1], loop,
                           jnp.full((ep_pad - E,), N, edge_index.dtype)])
    posp = jnp.pad(pos, ((0, _NP - N), (0, 5)))
    xp = jnp.pad(attr, ((0, _NP - N), (0, 5)))
    x = _multi_head_conv(xp, posp, src, dst, params['pt1'], N)
    x = _pos_attn(x, posp, params['attn'], N)
    x = _multi_head_conv(x, posp, src, dst, params['pt2'], N)
    return x[:N]


# speedup vs baseline: 1.1543x; 1.0346x over previous
import functools
import jax
import jax.numpy as jnp
import numpy as np
from jax import lax
from jax.experimental import pallas as pl
from jax.experimental.pallas import tpu as pltpu
from jax.experimental.pallas import tpu_sc as plsc

_NS = 16
_NC = 2
_NP = 10240
_EB = 64


def _edge_sc_body(n_rounds, src_h, dst_h, ax_h, pd_h,
                  st_out,
                  ids0, ids1, idd0, idd1, dst0, dst1,
                  ax0, ax1, pd0, pd1, st0, st_sh, sem0, sem1):
    c = lax.axis_index("c")
    sid = lax.axis_index("s")
    ep_pad = src_h.shape[0]
    per_tile = ep_pad // _NS
    n_chunks = per_tile // _EB
    rows_per = _NP // _NS
    rows0 = sid * rows_per
    banks = ((ids0, idd0, dst0, ax0, pd0, st0, sem0),
             (ids1, idd1, dst1, ax1, pd1, st0, sem1))

    def _load_idx(chunk, g, bk):
        ids, idd, dstv = bk[0], bk[1], bk[2]
        base = sid * per_tile + chunk * _EB
        pltpu.sync_copy(src_h.at[pl.ds(base, _EB)], ids)
        pltpu.sync_copy(dst_h.at[pl.ds(base, _EB)], dstv)
        for j in range(_EB // 16):
            sl = pl.ds(j * 16, 16)
            ids[sl] = ids[sl] + g * _NP
            idd[sl] = dstv[sl] + g * _NP

    def _issue(bk):
        pltpu.async_copy(ax_h.at[bk[0]], bk[3], bk[6])
        pltpu.async_copy(pd_h.at[bk[1]], bk[4], bk[6])

    def _wait(bk):
        pltpu.make_async_copy(ax_h.at[bk[0]], bk[3], bk[6]).wait()
        pltpu.make_async_copy(pd_h.at[bk[1]], bk[4], bk[6]).wait()

    for r in range(n_rounds):
        g = c * n_rounds + r
        def _z_row(i, _):
            for j in range(8):
                st0[i, pl.ds(j * 16, 16)] = jnp.zeros((16,), jnp.float32)
            return 0
        lax.fori_loop(0, _EB, _z_row, 0)
        for z in range(rows_per // _EB):
            pltpu.sync_copy(st0, st_sh.at[pl.ds(rows0 + z * _EB, _EB)])
        plsc.subcore_barrier()

        _load_idx(0, g, banks[0])
        _issue(banks[0])

        def _pair(ci, _):
            for b in range(2):
                bk = banks[b]
                nb = banks[1 - b]
                chunk = 2 * ci + b

                @pl.when(chunk + 1 < n_chunks)
                def _():
                    _load_idx(chunk + 1, g, nb)
                    _issue(nb)
                _wait(bk)
                ax_v, pd_v, st_v = bk[3], bk[4], bk[5]

                def _row(q, _):
                    for j in range(4):
                        sl = pl.ds(j * 16, 16)
                        sl2 = pl.ds(64 + j * 16, 16)
                        ee = jnp.exp(ax_v[q, sl] + pd_v[q, sl])
                        st_v[q, sl] = ee
                        st_v[q, sl2] = ee * (ax_v[q, sl2] + pd_v[q, sl2])
                    return 0
                lax.fori_loop(0, _EB, _row, 0)
                pltpu.sync_copy(st_v, st_sh.at[bk[2]], add=True)
            return 0
        lax.fori_loop(0, n_chunks // 2, _pair, 0)
        plsc.subcore_barrier()

        out0 = g * _NP + rows0
        pltpu.sync_copy(st_sh.at[pl.ds(rows0, rows_per)], st_out.at[pl.ds(out0, rows_per)])


def _edge_sc(src_p, dst_p, ax_flat, pd_flat, n_rounds):
    n_g = _NC * n_rounds
    mesh = plsc.VectorSubcoreMesh(core_axis_name="c", subcore_axis_name="s")
    f = pl.kernel(
        functools.partial(_edge_sc_body, n_rounds),
        out_type=jax.ShapeDtypeStruct((n_g * _NP, 128), jnp.float32),
        mesh=mesh,
        compiler_params=pltpu.CompilerParams(needs_layout_passes=False),
        scratch_types=(
            [pltpu.VMEM((_EB,), jnp.int32) for _ in range(6)]
            + [pltpu.VMEM((_EB, 128), jnp.float32) for _ in range(5)]
            + [pltpu.VMEM_SHARED((_NP, 128), jnp.float32),
               pltpu.SemaphoreType.DMA, pltpu.SemaphoreType.DMA]
        ),
    )
    return f(src_p, dst_p, ax_flat, pd_flat)


_BN = 512
_NBLK = _NP // _BN


def _dot(a, b):
    return jax.lax.dot_general(a, b, (((1,), (0,)), ((), ())),
                               preferred_element_type=jnp.float32)


def _ln(x, g, b, eps=1e-5):
    mu = jnp.mean(x, axis=-1, keepdims=True)
    var = jnp.mean((x - mu) ** 2, axis=-1, keepdims=True)
    return (x - mu) / jnp.sqrt(var + eps) * g + b


def _conv_max_body(x_r, p_r, wsrc_r, wpos_r, bpos_r, m_o, rm):
    i = pl.program_id(1)
    P = _dot(p_r[...], wpos_r[0])
    A = -_dot(x_r[...], wsrc_r[0]) - P
    PB = P + bpos_r[0]
    ca = jnp.max(A, axis=0, keepdims=True)
    cp = jnp.max(PB, axis=0, keepdims=True)

    @pl.when(i == 0)
    def _():
        rm[0:1] = ca
        rm[1:2] = cp

    @pl.when(i > 0)
    def _():
        rm[0:1] = jnp.maximum(rm[0:1], ca)
        rm[1:2] = jnp.maximum(rm[1:2], cp)

    @pl.when(i == _NBLK - 1)
    def _():
        m_o[0] = rm[0:1] + rm[1:2]


def _conv_pre_body(x_r, p_r, wsrc_r, wlin_r, blin_r, wpos_r, bpos_r, m_r,
                   ax_o, pd_o):
    x = x_r[...]
    p3 = p_r[...]
    P = _dot(p3, wpos_r[0])
    A = -_dot(x, wsrc_r[0]) - P - m_r[0]
    XV = _dot(x, wlin_r[0]) + blin_r[0] - P
    PB = P + bpos_r[0]
    ax_o[0] = jnp.concatenate([A, XV], axis=-1)
    pd_o[0] = jnp.concatenate([PB, PB], axis=-1)


def _conv_pre(xp, posp, wsrc_g, wlin_g, blin_g, wpos_g, bpos_g):
    F = xp.shape[1]
    n_g = wsrc_g.shape[0]
    m = pl.pallas_call(
        _conv_max_body,
        grid=(n_g, _NBLK),
        in_specs=[
            pl.BlockSpec((_BN, F), lambda g, i: (i, 0)),
            pl.BlockSpec((_BN, 8), lambda g, i: (i, 0)),
            pl.BlockSpec((1, F, 64), lambda g, i: (g, 0, 0)),
            pl.BlockSpec((1, 8, 64), lambda g, i: (g, 0, 0)),
            pl.BlockSpec((1, 1, 64), lambda g, i: (g, 0, 0)),
        ],
        out_specs=pl.BlockSpec((1, 1, 64), lambda g, i: (g, 0, 0)),
        out_shape=jax.ShapeDtypeStruct((n_g, 1, 64), jnp.float32),
        scratch_shapes=[pltpu.VMEM((2, 64), jnp.float32)],
    )(xp, posp, wsrc_g, wpos_g, bpos_g)
    ax, pd = pl.pallas_call(
        _conv_pre_body,
        grid=(n_g, _NBLK),
        in_specs=[
            pl.BlockSpec((_BN, F), lambda g, i: (i, 0)),
            pl.BlockSpec((_BN, 8), lambda g, i: (i, 0)),
            pl.BlockSpec((1, F, 64), lambda g, i: (g, 0, 0)),
            pl.BlockSpec((1, F, 64), lambda g, i: (g, 0, 0)),
            pl.BlockSpec((1, 1, 64), lambda g, i: (g, 0, 0)),
            pl.BlockSpec((1, 8, 64), lambda g, i: (g, 0, 0)),
            pl.BlockSpec((1, 1, 64), lambda g, i: (g, 0, 0)),
            pl.BlockSpec((1, 1, 64), lambda g, i: (g, 0, 0)),
        ],
        out_specs=[
            pl.BlockSpec((1, _BN, 128), lambda g, i: (g, i, 0)),
            pl.BlockSpec((1, _BN, 128), lambda g, i: (g, i, 0)),
        ],
        out_shape=[
            jax.ShapeDtypeStruct((n_g, _NP, 128), jnp.float32),
            jax.ShapeDtypeStruct((n_g, _NP, 128), jnp.float32),
        ],
    )(xp, posp, wsrc_g, wlin_g, blin_g, wpos_g, bpos_g, m)
    return ax, pd


def _conv_post_body(st_r, w1_r, b1_r, w2_r, b2_r, y_o, *, n_g):
    st = st_r[...]
    acc = None
    for g in range(n_g):
        S = st[g, :, :64]
        T = st[g, :, 64:]
        R = T / (S + 1e-16)
        pa = _dot(R, w1_r[pl.ds(g * 64, 64), :])
        acc = pa if acc is None else acc + pa
    h = jax.nn.relu(acc + b1_r[...])
    y_o[...] = _dot(h, w2_r[...]) + b2_r[...]


def _conv_post(st, w1, b1, w2, b2):
    n_g, _, _ = st.shape
    Hc, Ho = w2.shape
    return pl.pallas_call(
        functools.partial(_conv_post_body, n_g=n_g),
        grid=(_NBLK,),
        in_specs=[
            pl.BlockSpec((n_g, _BN, 128), lambda i: (0, i, 0)),
            pl.BlockSpec((Hc, Hc), lambda i: (0, 0)),
            pl.BlockSpec((1, Hc), lambda i: (0, 0)),
            pl.BlockSpec((Hc, Ho), lambda i: (0, 0)),
            pl.BlockSpec((1, Ho), lambda i: (0, 0)),
        ],
        out_specs=pl.BlockSpec((_BN, Ho), lambda i: (i, 0)),
        out_shape=jax.ShapeDtypeStruct((_NP, Ho), jnp.float32),
    )(st, w1, b1, w2, b2)


def _attn_pre_body(x_r, p_r, g1_r, b1_r, pe1w_r, pe1b_r, pe2w_r, pe2b_r,
                   wct_r, wcb_r, bc_r, wq_r, bq_r, wk_r, bk_r, wv_r, bv_r,
                   q_o, k_o, v_o):
    x = x_r[...]
    xn = _ln(x, g1_r[...], b1_r[...])
    pe = jax.nn.relu(_dot(p_r[...], pe1w_r[...]) + pe1b_r[...])
    pe = _dot(pe, pe2w_r[...]) + pe2b_r[...]
    xc = _dot(xn, wct_r[...]) + _dot(pe, wcb_r[...]) + bc_r[...]
    q_o[...] = _dot(xc, wq_r[...]) + bq_r[...]
    k_o[...] = _dot(xc, wk_r[...]) + bk_r[...]
    v_o[...] = _dot(xc, wv_r[...]) + bv_r[...]


def _attn_pre(xp, posp, p):
    dim = xp.shape[1]
    pe_d = p['pe2']['W'].shape[1]
    wct = p['comb']['W'][:dim]
    wcb = p['comb']['W'][dim:]
    args = (xp, posp, p['g1'][None], p['b1'][None],
            jnp.pad(p['pe1']['W'], ((0, 5), (0, 0))), p['pe1']['b'][None],
            p['pe2']['W'], p['pe2']['b'][None],
            wct, wcb, p['comb']['b'][None],
            p['q']['W'], p['q']['b'][None], p['k']['W'], p['k']['b'][None],
            p['v']['W'], p['v']['b'][None])
    specs = [
        pl.BlockSpec((_BN, dim), lambda i: (i, 0)),
        pl.BlockSpec((_BN, 8), lambda i: (i, 0)),
        pl.BlockSpec((1, dim), lambda i: (0, 0)),
        pl.BlockSpec((1, dim), lambda i: (0, 0)),
        pl.BlockSpec((8, pe_d), lambda i: (0, 0)),
        pl.BlockSpec((1, pe_d), lambda i: (0, 0)),
        pl.BlockSpec((pe_d, pe_d), lambda i: (0, 0)),
        pl.BlockSpec((1, pe_d), lambda i: (0, 0)),
        pl.BlockSpec((dim, dim), lambda i: (0, 0)),
        pl.BlockSpec((pe_d, dim), lambda i: (0, 0)),
        pl.BlockSpec((1, dim), lambda i: (0, 0)),
        pl.BlockSpec((dim, dim), lambda i: (0, 0)),
        pl.BlockSpec((1, dim), lambda i: (0, 0)),
        pl.BlockSpec((dim, dim), lambda i: (0, 0)),
        pl.BlockSpec((1, dim), lambda i: (0, 0)),
        pl.BlockSpec((dim, dim), lambda i: (0, 0)),
        pl.BlockSpec((1, dim), lambda i: (0, 0)),
    ]
    out = jax.ShapeDtypeStruct((_NP, dim), jnp.float32)
    return pl.pallas_call(
        _attn_pre_body,
        grid=(_NBLK,),
        in_specs=specs,
        out_specs=[pl.BlockSpec((_BN, dim), lambda i: (i, 0))] * 3,
        out_shape=[out, out, out],
    )(*args)


def _attn_body(q_ref, k_ref, v_ref, o_ref, *, n_valid, scale):
    q = (q_ref[0] * scale).astype(jnp.bfloat16)
    k = k_ref[0].astype(jnp.bfloat16)
    v = v_ref[0].astype(jnp.bfloat16)
    s = jax.lax.dot_general(q, k, (((1,), (1,)), ((), ())),
                            preferred_element_type=jnp.float32)
    col = jax.lax.broadcasted_iota(jnp.int32, s.shape, 1)
    s = jnp.where(col < n_valid, s, -1e30)
    m = jnp.max(s, axis=1, keepdims=True)
    pf = jnp.exp(s - m)
    den = jnp.sum(pf, axis=1, keepdims=True)
    p = pf.astype(jnp.bfloat16)
    o = jax.lax.dot_general(p, v, (((1,), (0,)), ((), ())),
                            preferred_element_type=jnp.float32)
    o_ref[0] = o / den


def _attention(q, k, v, n_valid, num_heads=4, bq=256):
    _, Np, hd = q.shape
    scale = 1.0 / np.sqrt(hd)
    return pl.pallas_call(
        functools.partial(_attn_body, n_valid=n_valid, scale=scale),
        grid=(num_heads, Np // bq),
        in_specs=[
            pl.BlockSpec((1, bq, hd), lambda h, i: (h, i, 0)),
            pl.BlockSpec((1, Np, hd), lambda h, i: (h, 0, 0)),
            pl.BlockSpec((1, Np, hd), lambda h, i: (h, 0, 0)),
        ],
        out_specs=pl.BlockSpec((1, bq, hd), lambda h, i: (h, i, 0)),
        out_shape=jax.ShapeDtypeStruct((num_heads, Np, hd), jnp.float32),
    )(q, k, v)


def _attn_post_body(o_r, x_r, wo_r, bo_r, g2_r, b2_r, y_o):
    out = _dot(o_r[...], wo_r[...]) + bo_r[...] + x_r[...]
    y_o[...] = _ln(out, g2_r[...], b2_r[...])


def _attn_post(o, xp, p):
    dim = xp.shape[1]
    return pl.pallas_call(
        _attn_post_body,
        grid=(_NBLK,),
        in_specs=[
            pl.BlockSpec((_BN, dim), lambda i: (i, 0)),
            pl.BlockSpec((_BN, dim), lambda i: (i, 0)),
            pl.BlockSpec((dim, dim), lambda i: (0, 0)),
            pl.BlockSpec((1, dim), lambda i: (0, 0)),
            pl.BlockSpec((1, dim), lambda i: (0, 0)),
            pl.BlockSpec((1, dim), lambda i: (0, 0)),
        ],
        out_specs=pl.BlockSpec((_BN, dim), lambda i: (i, 0)),
        out_shape=jax.ShapeDtypeStruct((_NP, dim), jnp.float32),
    )(o, xp, p['o']['W'], p['o']['b'][None], p['g2'][None], p['b2'][None])


def _multi_head_conv(xp, posp, src, dst, p, n_valid):
    Wsrc = jnp.concatenate([hp['src']['W'] for hp in p['heads']], axis=1)
    Wlin = jnp.concatenate([hp['lin']['W'] for hp in p['heads']], axis=1)
    blin = jnp.concatenate([hp['lin']['b'] for hp in p['heads']], axis=0)
    Wpos = jnp.concatenate([hp['pos']['W'] for hp in p['heads']], axis=1)
    bpos = jnp.concatenate([hp['pos']['b'] for hp in p['heads']], axis=0)
    F = xp.shape[1]
    Hc = Wsrc.shape[1]
    n_g = Hc // 64
    n_rounds = n_g // _NC
    if Wsrc.shape[0] != F:
        Wsrc = jnp.pad(Wsrc, ((0, F - Wsrc.shape[0]), (0, 0)))
        Wlin = jnp.pad(Wlin, ((0, F - Wlin.shape[0]), (0, 0)))

    def _grp(w):
        return w.reshape(w.shape[0], n_g, 64).transpose(1, 0, 2)
    ax, pd = _conv_pre(xp, posp, _grp(Wsrc), _grp(Wlin),
                       blin.reshape(n_g, 1, 64),
                       _grp(jnp.pad(Wpos, ((0, 5), (0, 0)))),
                       bpos.reshape(n_g, 1, 64))
    ST = _edge_sc(src, dst, ax.reshape(n_g * _NP, 128),
                  pd.reshape(n_g * _NP, 128), n_rounds)
    return _conv_post(ST.reshape(n_g, _NP, 128), p['p1']['W'],
                      p['p1']['b'][None], p['p2']['W'], p['p2']['b'][None])


def _pos_attn(xp, posp, p, n_valid, num_heads=4):
    q, k, v = _attn_pre(xp, posp, p)
    Np, dim = q.shape
    hd = dim // num_heads

    def _heads(t):
        return t.reshape(Np, num_heads, hd).transpose(1, 0, 2)
    o = _attention(_heads(q), _heads(k), _heads(v), n_valid, num_heads)
    o = o.transpose(1, 0, 2).reshape(Np, dim)
    return _attn_post(o, xp, p)


def kernel(pos, attr, edge_index, params):
    N = pos.shape[0]
    loop = jnp.arange(N, dtype=edge_index.dtype)
    E = edge_index.shape[1] + N
    ep_pad = -(-E // (_NS * _EB * 2)) * (_NS * _EB * 2)
    src = jnp.concatenate([edge_index[0], loop,
                           jnp.zeros((ep_pad - E,), edge_index.dtype)])
    dst = jnp.concatenate([edge_index[1], loop,
                           jnp.full((ep_pad - E,), N, edge_index.dtype)])
    posp = jnp.pad(pos, ((0, _NP - N), (0, 5)))
    xp = jnp.pad(attr, ((0, _NP - N), (0, 5)))
    x = _multi_head_conv(xp, posp, src, dst, params['pt1'], N)
    x = _pos_attn(x, posp, params['attn'], N)
    x = _multi_head_conv(x, posp, src, dst, params['pt2'], N)
    return x[:N]
